# baseline (speedup 1.0000x reference)
import functools

import jax
import jax.numpy as jnp
from jax import lax
from jax.experimental import pallas as pl
from jax.experimental.pallas import tpu as pltpu
from jax.experimental.pallas import tpu_sc as plsc

VOC = 1000000
DIM = 64
MAXLEN = 512
SEGN = 2
B = 1024
L = 200
EPS = 1e-06

N = B * L
NC = 2
NS = 16
NW = NC * NS
PER_W = N // NW
CR = 128
CNK = PER_W // CR
NPAIR = CNK // 2
GRP = CR // 16
NV = DIM // 16
PLINES = MAXLEN // 2


def _rsqrt(v):
    i = lax.bitcast_convert_type(v, jnp.int32)
    i = jnp.int32(0x5F3759DF) - lax.shift_right_logical(i, 1)
    y = lax.bitcast_convert_type(i, jnp.float32)
    for _ in range(3):
        y = y * (1.5 - 0.5 * v * y * y)
    return y


def _body(emb_h, pose_h, sege_h, gam_h, bet_h,
          xl_h, xp_h, s_h, pll_h, pp_h, out_h,
          xidx0, xidx1, pidx0, pidx1,
          sidx0, sidx1, xpar0, xpar1, ppar0, ppar1,
          re0, re1, ro_v,
          fused_v, ht_v, gam_v, bet_v, seg_v,
          seme0, seme1):
    cid = lax.axis_index("c")
    sid = lax.axis_index("s")
    wid = sid * NC + cid

    xidx = [xidx0, xidx1]
    pidx = [pidx0, pidx1]
    sidx = [sidx0, sidx1]
    xpar = [xpar0, xpar1]
    ppar = [ppar0, ppar1]
    re = [re0, re1]
    ro = [ro_v, ro_v]
    seme = [seme0, seme1]

    pltpu.sync_copy(gam_h, gam_v)
    pltpu.sync_copy(bet_h, bet_v)
    pltpu.sync_copy(sege_h, seg_v)

    g = [gam_v[pl.ds(16 * k, 16)] for k in range(NV)]
    b = [bet_v[pl.ds(16 * k, 16)] for k in range(NV)]
    sg = [[seg_v[pl.ds(s * DIM + 16 * k, 16)] for k in range(NV)]
          for s in range(SEGN)]

    lanes = lax.iota(jnp.int32, 16)

    pltpu.sync_copy(pose_h, fused_v.at[0])
    pltpu.sync_copy(pose_h, fused_v.at[1])

    @plsc.parallel_loop(0, PLINES)
    def _(r):
        for s in range(SEGN):
            for t in range(2):
                for k in range(NV):
                    off = t * DIM + 16 * k
                    fused_v[s, r, pl.ds(off, 16)] = \
                        fused_v[s, r, pl.ds(off, 16)] + sg[s][k]

    def issue(c, bs):
        rb = (wid * CNK + c) * CR
        pltpu.sync_copy(xl_h.at[pl.ds(rb, CR)], xidx[bs])
        pltpu.sync_copy(pll_h.at[pl.ds(rb, CR)], pidx[bs])
        pltpu.sync_copy(s_h.at[pl.ds(rb, CR)], sidx[bs])
        pltpu.sync_copy(xp_h.at[pl.ds(rb, CR)], xpar[bs])
        pltpu.sync_copy(pp_h.at[pl.ds(rb, CR)], ppar[bs])
        pltpu.async_copy(emb_h.at[xidx[bs]], re[bs], seme[bs])

    def wait_gathers(bs):
        pltpu.make_async_copy(emb_h.at[xidx[bs]], re[bs],
                              seme[bs]).wait()

    def compute(bs):
        re_b, ro_b = re[bs], ro[bs]
        sidx_b, xpar_b, ppar_b, pidx_b = \
            sidx[bs], xpar[bs], ppar[bs], pidx[bs]

        @plsc.parallel_loop(0, GRP)
        def _(gi):
            rows = gi * 16 + lanes
            sidx_g = sidx_b[pl.ds(gi * 16, 16)]
            xo = xpar_b[pl.ds(gi * 16, 16)] * 64
            po = ppar_b[pl.ds(gi * 16, 16)] * 64
            pline = pidx_b[pl.ds(gi * 16, 16)]
            acc = jnp.zeros((16,), jnp.float32)
            acc2 = jnp.zeros((16,), jnp.float32)
            for c in range(DIM):
                e = plsc.load_gather(re_b, [rows, xo + c])
                f = plsc.load_gather(fused_v, [sidx_g, pline, po + c])
                h = e + f
                ht_v[pl.ds((gi * DIM + c) * 16, 16)] = h
                acc = acc + h
                acc2 = acc2 + h * h
            mu = acc * (1.0 / DIM)
            var = acc2 * (1.0 / DIM) - mu * mu
            rv = _rsqrt(var + EPS)
            rbase = rows * DIM
            for c in range(DIM):
                h = ht_v[pl.ds((gi * DIM + c) * 16, 16)]
                k, l = divmod(c, 16)
                val = (h - mu) * rv * g[k][l] + b[k][l]
                plsc.store_scatter(ro_b, [rbase + c], val)

    def pair_body(i, carry):
        wait_gathers(0)
        issue(2 * i + 1, 1)
        compute(0)
        pltpu.sync_copy(
            ro[0], out_h.at[pl.ds((wid * CNK + 2 * i) * CR * DIM, CR * DIM)])

        wait_gathers(1)

        @pl.when(i < NPAIR - 1)
        def _():
            issue(2 * i + 2, 0)

        compute(1)
        pltpu.sync_copy(
            ro[1],
            out_h.at[pl.ds((wid * CNK + 2 * i + 1) * CR * DIM, CR * DIM)])
        return carry

    issue(0, 0)
    lax.fori_loop(0, NPAIR, pair_body, 0, unroll=False)


_emb_ln = functools.partial(
    pl.kernel,
    out_type=jax.ShapeDtypeStruct((N * DIM,), jnp.float32),
    mesh=plsc.VectorSubcoreMesh(core_axis_name="c", subcore_axis_name="s"),
    compiler_params=pltpu.CompilerParams(use_tc_tiling_on_sc=True,
                                         needs_layout_passes=False),
    scratch_types=[
        pltpu.VMEM((CR,), jnp.int32),
        pltpu.VMEM((CR,), jnp.int32),
        pltpu.VMEM((CR,), jnp.int32),
        pltpu.VMEM((CR,), jnp.int32),
        pltpu.VMEM((CR,), jnp.int32),
        pltpu.VMEM((CR,), jnp.int32),
        pltpu.VMEM((CR,), jnp.int32),
        pltpu.VMEM((CR,), jnp.int32),
        pltpu.VMEM((CR,), jnp.int32),
        pltpu.VMEM((CR,), jnp.int32),
        pltpu.VMEM((CR, 2 * DIM), jnp.float32),
        pltpu.VMEM((CR, 2 * DIM), jnp.float32),
        pltpu.VMEM((CR * DIM,), jnp.float32),
        pltpu.VMEM((SEGN, PLINES, 2 * DIM), jnp.float32),
        pltpu.VMEM((GRP * DIM * 16,), jnp.float32),
        pltpu.VMEM((DIM,), jnp.float32),
        pltpu.VMEM((DIM,), jnp.float32),
        pltpu.VMEM((SEGN * DIM,), jnp.float32),
        pltpu.SemaphoreType.DMA,
        pltpu.SemaphoreType.DMA,
    ],
)(_body)


@jax.jit
def kernel(emb, posemb, segemb, gamma, beta, x, seg, pos):
    emb128 = emb.reshape(VOC // 2, 2 * DIM)
    pose128 = posemb.reshape(MAXLEN // 2, 2 * DIM)
    xl = (x >> 1).reshape(N)
    xp = (x & 1).reshape(N)
    pll = (pos >> 1).reshape(N)
    pp = (pos & 1).reshape(N)
    sf = seg.reshape(N)
    out = _emb_ln(emb128, pose128, segemb.reshape(SEGN * DIM),
                  gamma, beta, xl, xp, sf, pll, pp)
    return out.reshape(B, L, DIM)


# speedup vs baseline: 1.4621x; 1.4621x over previous
import functools

import jax
import jax.numpy as jnp
from jax import lax
from jax.experimental import pallas as pl
from jax.experimental.pallas import tpu as pltpu
from jax.experimental.pallas import tpu_sc as plsc

VOC = 1000000
DIM = 64
MAXLEN = 512
SEGN = 2
B = 1024
L = 200
EPS = 1e-06

N = B * L
NC = 2
NS = 16
NW = NC * NS
PER_W = N // NW
CR = 128
CNK = PER_W // CR
NPAIR = CNK // 2
GRP = CR // 16
NV = DIM // 16
PLINES = MAXLEN // 2


def _rsqrt(v):
    i = lax.bitcast_convert_type(v, jnp.int32)
    i = jnp.int32(0x5F3759DF) - lax.shift_right_logical(i, 1)
    y = lax.bitcast_convert_type(i, jnp.float32)
    for _ in range(3):
        y = y * (1.5 - 0.5 * v * y * y)
    return y


def _body(emb_h, pose_h, sege_h, gam_h, bet_h,
          xl_h, xp_h, s_h, pll_h, pp_h, out_h,
          xidx0, xidx1, pidx0, pidx1,
          sidx0, sidx1, xpar0, xpar1, ppar0, ppar1,
          re0, re1, ro_v,
          fused_v, ht_v, gam_v, bet_v, seg_v, grot_v, brot_v,
          seme0, seme1):
    cid = lax.axis_index("c")
    sid = lax.axis_index("s")
    wid = sid * NC + cid

    xidx = [xidx0, xidx1]
    pidx = [pidx0, pidx1]
    sidx = [sidx0, sidx1]
    xpar = [xpar0, xpar1]
    ppar = [ppar0, ppar1]
    re = [re0, re1]
    ro = [ro_v, ro_v]
    seme = [seme0, seme1]

    pltpu.sync_copy(gam_h, gam_v)
    pltpu.sync_copy(bet_h, bet_v)
    pltpu.sync_copy(sege_h, seg_v)

    g = [gam_v[pl.ds(16 * k, 16)] for k in range(NV)]
    b = [bet_v[pl.ds(16 * k, 16)] for k in range(NV)]
    sg = [[seg_v[pl.ds(s * DIM + 16 * k, 16)] for k in range(NV)]
          for s in range(SEGN)]

    lanes = lax.iota(jnp.int32, 16)
    lanes128 = lanes * 128
    lanes64 = lanes * DIM

    for c in range(DIM):
        cv = (lanes + c) & 63
        grot_v[pl.ds(c * 16, 16)] = plsc.load_gather(gam_v, [cv])
        brot_v[pl.ds(c * 16, 16)] = plsc.load_gather(bet_v, [cv])

    pltpu.sync_copy(pose_h, fused_v.at[0])
    pltpu.sync_copy(pose_h, fused_v.at[1])

    @plsc.parallel_loop(0, PLINES)
    def _(r):
        for s in range(SEGN):
            for t in range(2):
                for k in range(NV):
                    off = t * DIM + 16 * k
                    fused_v[s, r, pl.ds(off, 16)] = \
                        fused_v[s, r, pl.ds(off, 16)] + sg[s][k]

    def issue(c, bs):
        rb = (wid * CNK + c) * CR
        pltpu.sync_copy(xl_h.at[pl.ds(rb, CR)], xidx[bs])
        pltpu.sync_copy(pll_h.at[pl.ds(rb, CR)], pidx[bs])
        pltpu.sync_copy(s_h.at[pl.ds(rb, CR)], sidx[bs])
        pltpu.sync_copy(xp_h.at[pl.ds(rb, CR)], xpar[bs])
        pltpu.sync_copy(pp_h.at[pl.ds(rb, CR)], ppar[bs])
        pltpu.async_copy(emb_h.at[xidx[bs]], re[bs], seme[bs])

    def wait_gathers(bs):
        pltpu.make_async_copy(emb_h.at[xidx[bs]], re[bs],
                              seme[bs]).wait()

    def compute(bs):
        re_b, ro_b = re[bs], ro[bs]
        sidx_b, xpar_b, ppar_b, pidx_b = \
            sidx[bs], xpar[bs], ppar[bs], pidx[bs]

        @plsc.parallel_loop(0, GRP)
        def _(gi):
            rows = gi * 16 + lanes
            sidx_g = sidx_b[pl.ds(gi * 16, 16)]
            xo = xpar_b[pl.ds(gi * 16, 16)] * 64
            po = ppar_b[pl.ds(gi * 16, 16)] * 64
            pline = pidx_b[pl.ds(gi * 16, 16)]
            acc = jnp.zeros((16,), jnp.float32)
            acc2 = jnp.zeros((16,), jnp.float32)
            for c in range(DIM):
                cv = (lanes + c) & 63
                e = plsc.load_gather(re_b, [rows, xo + cv])
                f = plsc.load_gather(fused_v, [sidx_g, pline, po + cv])
                h = e + f
                ht_v[pl.ds((gi * DIM + c) * 16, 16)] = h
                acc = acc + h
                acc2 = acc2 + h * h
            mu = acc * (1.0 / DIM)
            var = acc2 * (1.0 / DIM) - mu * mu
            rv = _rsqrt(var + EPS)
            obase = gi * (16 * DIM) + lanes64
            for c in range(DIM):
                cv = (lanes + c) & 63
                h = ht_v[pl.ds((gi * DIM + c) * 16, 16)]
                gc = grot_v[pl.ds(c * 16, 16)]
                bc = brot_v[pl.ds(c * 16, 16)]
                val = (h - mu) * rv * gc + bc
                plsc.store_scatter(ro_b, [obase + cv], val)

    def pair_body(i, carry):
        wait_gathers(0)
        issue(2 * i + 1, 1)
        compute(0)
        pltpu.sync_copy(
            ro[0], out_h.at[pl.ds((wid * CNK + 2 * i) * CR * DIM, CR * DIM)])

        wait_gathers(1)

        @pl.when(i < NPAIR - 1)
        def _():
            issue(2 * i + 2, 0)

        compute(1)
        pltpu.sync_copy(
            ro[1],
            out_h.at[pl.ds((wid * CNK + 2 * i + 1) * CR * DIM, CR * DIM)])
        return carry

    issue(0, 0)
    lax.fori_loop(0, NPAIR, pair_body, 0, unroll=False)


_emb_ln = functools.partial(
    pl.kernel,
    out_type=jax.ShapeDtypeStruct((N * DIM,), jnp.float32),
    mesh=plsc.VectorSubcoreMesh(core_axis_name="c", subcore_axis_name="s"),
    compiler_params=pltpu.CompilerParams(use_tc_tiling_on_sc=True,
                                         needs_layout_passes=False),
    scratch_types=[
        pltpu.VMEM((CR,), jnp.int32),
        pltpu.VMEM((CR,), jnp.int32),
        pltpu.VMEM((CR,), jnp.int32),
        pltpu.VMEM((CR,), jnp.int32),
        pltpu.VMEM((CR,), jnp.int32),
        pltpu.VMEM((CR,), jnp.int32),
        pltpu.VMEM((CR,), jnp.int32),
        pltpu.VMEM((CR,), jnp.int32),
        pltpu.VMEM((CR,), jnp.int32),
        pltpu.VMEM((CR,), jnp.int32),
        pltpu.VMEM((CR, 2 * DIM), jnp.float32),
        pltpu.VMEM((CR, 2 * DIM), jnp.float32),
        pltpu.VMEM((CR * DIM,), jnp.float32),
        pltpu.VMEM((SEGN, PLINES, 2 * DIM), jnp.float32),
        pltpu.VMEM((GRP * DIM * 16,), jnp.float32),
        pltpu.VMEM((DIM,), jnp.float32),
        pltpu.VMEM((DIM,), jnp.float32),
        pltpu.VMEM((SEGN * DIM,), jnp.float32),
        pltpu.VMEM((DIM * 16,), jnp.float32),
        pltpu.VMEM((DIM * 16,), jnp.float32),
        pltpu.SemaphoreType.DMA,
        pltpu.SemaphoreType.DMA,
    ],
)(_body)


@jax.jit
def kernel(emb, posemb, segemb, gamma, beta, x, seg, pos):
    emb128 = emb.reshape(VOC // 2, 2 * DIM)
    pose128 = posemb.reshape(MAXLEN // 2, 2 * DIM)
    xl = (x >> 1).reshape(N)
    xp = (x & 1).reshape(N)
    pll = (pos >> 1).reshape(N)
    pp = (pos & 1).reshape(N)
    sf = seg.reshape(N)
    out = _emb_ln(emb128, pose128, segemb.reshape(SEGN * DIM),
                  gamma, beta, xl, xp, sf, pll, pp)
    return out.reshape(B, L, DIM)


# speedup vs baseline: 1.4673x; 1.0036x over previous
import functools

import jax
import jax.numpy as jnp
from jax import lax
from jax.experimental import pallas as pl
from jax.experimental.pallas import tpu as pltpu
from jax.experimental.pallas import tpu_sc as plsc

VOC = 1000000
DIM = 64
MAXLEN = 512
SEGN = 2
B = 1024
L = 200
EPS = 1e-06

N = B * L
NC = 2
NS = 16
NW = NC * NS
PER_W = N // NW
CR = 128
CNK = PER_W // CR
NPAIR = CNK // 2
GRP = CR // 16
NV = DIM // 16

_GDIMS = lax.GatherDimensionNumbers(
    offset_dims=(), collapsed_slice_dims=(0,), start_index_map=(0,))


def _allsum(v, perms):
    for p in perms:
        v = v + lax.gather(v, p, _GDIMS, (1,),
                           mode=lax.GatherScatterMode.PROMISE_IN_BOUNDS)
    return v


def _rsqrt(v):
    i = lax.bitcast_convert_type(v, jnp.int32)
    i = jnp.int32(0x5F3759DF) - lax.shift_right_logical(i, 1)
    y = lax.bitcast_convert_type(i, jnp.float32)
    for _ in range(3):
        y = y * (1.5 - 0.5 * v * y * y)
    return y


def _body(emb_h, pose_h, sege_h, gam_h, bet_h,
          xl_h, xp_h, s_h, pll_h, pp_h, out_h,
          xidx0, xidx1, pidx0, pidx1,
          sidx0, sidx1, xpar0, xpar1, ppar0, ppar1,
          re0, re1, rp0, rp1, ro_v,
          gam_v, bet_v, seg_v,
          seme0, seme1, semp0, semp1):
    cid = lax.axis_index("c")
    sid = lax.axis_index("s")
    wid = sid * NC + cid

    xidx = [xidx0, xidx1]
    pidx = [pidx0, pidx1]
    sidx = [sidx0, sidx1]
    xpar = [xpar0, xpar1]
    ppar = [ppar0, ppar1]
    re = [re0, re1]
    rp = [rp0, rp1]
    ro = [ro_v, ro_v]
    seme = [seme0, seme1]
    semp = [semp0, semp1]

    pltpu.sync_copy(gam_h, gam_v)
    pltpu.sync_copy(bet_h, bet_v)
    pltpu.sync_copy(sege_h, seg_v)

    g = [gam_v[pl.ds(16 * k, 16)] for k in range(NV)]
    b = [bet_v[pl.ds(16 * k, 16)] for k in range(NV)]
    s0 = [seg_v[pl.ds(16 * k, 16)] for k in range(NV)]
    sd = [seg_v[pl.ds(DIM + 16 * k, 16)] - s0[k] for k in range(NV)]

    lanes = lax.iota(jnp.int32, 16)
    perms = [(lanes ^ (1 << k))[:, None] for k in range(4)]

    def issue(c, bs):
        rb = (wid * CNK + c) * CR
        pltpu.sync_copy(xl_h.at[pl.ds(rb, CR)], xidx[bs])
        pltpu.sync_copy(pll_h.at[pl.ds(rb, CR)], pidx[bs])
        pltpu.sync_copy(s_h.at[pl.ds(rb, CR)], sidx[bs])
        pltpu.sync_copy(xp_h.at[pl.ds(rb, CR)], xpar[bs])
        pltpu.sync_copy(pp_h.at[pl.ds(rb, CR)], ppar[bs])
        pltpu.async_copy(emb_h.at[xidx[bs]], re[bs], seme[bs])
        pltpu.async_copy(pose_h.at[pidx[bs]], rp[bs], semp[bs])

    def wait_gathers(bs):
        pltpu.make_async_copy(emb_h.at[xidx[bs]], re[bs], seme[bs]).wait()
        pltpu.make_async_copy(pose_h.at[pidx[bs]], rp[bs], semp[bs]).wait()

    def compute(bs):
        re_b, rp_b, ro_b = re[bs], rp[bs], ro[bs]
        sidx_b, xpar_b, ppar_b = sidx[bs], xpar[bs], ppar[bs]

        @plsc.parallel_loop(0, GRP)
        def _(gi):
            sff = sidx_b[pl.ds(gi * 16, 16)].astype(jnp.float32)
            xff = xpar_b[pl.ds(gi * 16, 16)].astype(jnp.float32)
            pff = ppar_b[pl.ds(gi * 16, 16)].astype(jnp.float32)
            for j in range(16):
                i = gi * 16 + j
                sf = sff[j]
                xf = xff[j]
                pf = pff[j]
                hs = []
                for k in range(NV):
                    eL = re_b[i, pl.ds(16 * k, 16)]
                    eH = re_b[i, pl.ds(DIM + 16 * k, 16)]
                    pL = rp_b[i, pl.ds(16 * k, 16)]
                    pH = rp_b[i, pl.ds(DIM + 16 * k, 16)]
                    e = eL + xf * (eH - eL)
                    p = pL + pf * (pH - pL)
                    hs.append(e + p + s0[k] + sf * sd[k])
                sv = (hs[0] + hs[1]) + (hs[2] + hs[3])
                qv = (hs[0] * hs[0] + hs[1] * hs[1]) \
                    + (hs[2] * hs[2] + hs[3] * hs[3])
                mu = _allsum(sv, perms) * (1.0 / DIM)
                var = _allsum(qv, perms) * (1.0 / DIM) - mu * mu
                rv = _rsqrt(var + EPS)
                for k in range(NV):
                    ro_b[pl.ds(i * DIM + 16 * k, 16)] = \
                        (hs[k] - mu) * rv * g[k] + b[k]

    def pair_body(i, carry):
        wait_gathers(0)
        issue(2 * i + 1, 1)
        compute(0)
        pltpu.sync_copy(
            ro[0], out_h.at[pl.ds((wid * CNK + 2 * i) * CR * DIM, CR * DIM)])

        wait_gathers(1)

        @pl.when(i < NPAIR - 1)
        def _():
            issue(2 * i + 2, 0)

        compute(1)
        pltpu.sync_copy(
            ro[1],
            out_h.at[pl.ds((wid * CNK + 2 * i + 1) * CR * DIM, CR * DIM)])
        return carry

    issue(0, 0)
    lax.fori_loop(0, NPAIR, pair_body, 0, unroll=False)


_emb_ln = functools.partial(
    pl.kernel,
    out_type=jax.ShapeDtypeStruct((N * DIM,), jnp.float32),
    mesh=plsc.VectorSubcoreMesh(core_axis_name="c", subcore_axis_name="s"),
    compiler_params=pltpu.CompilerParams(use_tc_tiling_on_sc=True),
    scratch_types=[
        pltpu.VMEM((CR,), jnp.int32),
        pltpu.VMEM((CR,), jnp.int32),
        pltpu.VMEM((CR,), jnp.int32),
        pltpu.VMEM((CR,), jnp.int32),
        pltpu.VMEM((CR,), jnp.int32),
        pltpu.VMEM((CR,), jnp.int32),
        pltpu.VMEM((CR,), jnp.int32),
        pltpu.VMEM((CR,), jnp.int32),
        pltpu.VMEM((CR,), jnp.int32),
        pltpu.VMEM((CR,), jnp.int32),
        pltpu.VMEM((CR, 2 * DIM), jnp.float32),
        pltpu.VMEM((CR, 2 * DIM), jnp.float32),
        pltpu.VMEM((CR, 2 * DIM), jnp.float32),
        pltpu.VMEM((CR, 2 * DIM), jnp.float32),
        pltpu.VMEM((CR * DIM,), jnp.float32),
        pltpu.VMEM((DIM,), jnp.float32),
        pltpu.VMEM((DIM,), jnp.float32),
        pltpu.VMEM((SEGN * DIM,), jnp.float32),
        pltpu.SemaphoreType.DMA,
        pltpu.SemaphoreType.DMA,
        pltpu.SemaphoreType.DMA,
        pltpu.SemaphoreType.DMA,
    ],
)(_body)


@jax.jit
def kernel(emb, posemb, segemb, gamma, beta, x, seg, pos):
    emb128 = emb.reshape(VOC // 2, 2 * DIM)
    pose128 = posemb.reshape(MAXLEN // 2, 2 * DIM)
    xl = (x >> 1).reshape(N)
    xp = (x & 1).reshape(N)
    pll = (pos >> 1).reshape(N)
    pp = (pos & 1).reshape(N)
    sf = seg.reshape(N)
    out = _emb_ln(emb128, pose128, segemb.reshape(SEGN * DIM),
                  gamma, beta, xl, xp, sf, pll, pp)
    return out.reshape(B, L, DIM)


# speedup vs baseline: 2.0677x; 1.4091x over previous
import functools

import jax
import jax.numpy as jnp
from jax import lax
from jax.experimental import pallas as pl
from jax.experimental.pallas import tpu as pltpu
from jax.experimental.pallas import tpu_sc as plsc

VOC = 1000000
DIM = 64
MAXLEN = 512
SEGN = 2
B = 1024
L = 200
EPS = 1e-06

N = B * L
NC = 2
NS = 16
NW = NC * NS
PER_W = N // NW
CR = 128
CNK = PER_W // CR
NPAIR = CNK // 2
GRP = CR // 16
NV = DIM // 16
HV = VOC // 2

_GDIMS = lax.GatherDimensionNumbers(
    offset_dims=(), collapsed_slice_dims=(0,), start_index_map=(0,))


def _allsum(v, perms):
    for p in perms:
        v = v + lax.gather(v, p, _GDIMS, (1,),
                           mode=lax.GatherScatterMode.PROMISE_IN_BOUNDS)
    return v


def _rsqrt(v):
    i = lax.bitcast_convert_type(v, jnp.int32)
    i = jnp.int32(0x5F3759DF) - lax.shift_right_logical(i, 1)
    y = lax.bitcast_convert_type(i, jnp.float32)
    for _ in range(3):
        y = y * (1.5 - 0.5 * v * y * y)
    return y


def _body(emb_h, pose_h, sege_h, gam_h, bet_h,
          xg_h, s_h, p_h, out_h,
          xidx0, xidx1, pidx0, pidx1, sidx0, sidx1,
          re0, re1, rp0, rp1, ro_v,
          gam_v, bet_v, seg_v,
          seme0, seme1, semp0, semp1):
    cid = lax.axis_index("c")
    sid = lax.axis_index("s")
    wid = sid * NC + cid

    xidx = [xidx0, xidx1]
    pidx = [pidx0, pidx1]
    sidx = [sidx0, sidx1]
    re = [re0, re1]
    rp = [rp0, rp1]
    ro = [ro_v, ro_v]
    seme = [seme0, seme1]
    semp = [semp0, semp1]

    pltpu.sync_copy(gam_h, gam_v)
    pltpu.sync_copy(bet_h, bet_v)
    pltpu.sync_copy(sege_h, seg_v)

    g = [gam_v[pl.ds(16 * k, 16)] for k in range(NV)]
    b = [bet_v[pl.ds(16 * k, 16)] for k in range(NV)]
    s0 = [seg_v[pl.ds(16 * k, 16)] for k in range(NV)]
    sd = [seg_v[pl.ds(DIM + 16 * k, 16)] - s0[k] for k in range(NV)]

    lanes = lax.iota(jnp.int32, 16)
    perms = [(lanes ^ (1 << k))[:, None] for k in range(4)]

    def issue(c, bs):
        rb = (wid * CNK + c) * CR
        pltpu.sync_copy(xg_h.at[pl.ds(rb, CR)], xidx[bs])
        pltpu.sync_copy(p_h.at[pl.ds(rb, CR)], pidx[bs])
        pltpu.sync_copy(s_h.at[pl.ds(rb, CR)], sidx[bs])
        pltpu.async_copy(emb_h.at[xidx[bs]], re[bs], seme[bs])
        pltpu.async_copy(pose_h.at[pidx[bs]], rp[bs], semp[bs])

    def wait_gathers(bs):
        pltpu.make_async_copy(emb_h.at[xidx[bs]], re[bs], seme[bs]).wait()
        pltpu.make_async_copy(pose_h.at[pidx[bs]], rp[bs], semp[bs]).wait()

    def compute(bs):
        re_b, rp_b, ro_b = re[bs], rp[bs], ro[bs]
        sidx_b = sidx[bs]

        @plsc.parallel_loop(0, GRP)
        def _(gi):
            sff = sidx_b[pl.ds(gi * 16, 16)].astype(jnp.float32)
            for j in range(16):
                i = gi * 16 + j
                sf = sff[j]
                hs = []
                for k in range(NV):
                    e = re_b[i, pl.ds(16 * k, 16)]
                    p = rp_b[i, pl.ds(16 * k, 16)]
                    hs.append(e + p + s0[k] + sf * sd[k])
                sv = (hs[0] + hs[1]) + (hs[2] + hs[3])
                qv = (hs[0] * hs[0] + hs[1] * hs[1]) \
                    + (hs[2] * hs[2] + hs[3] * hs[3])
                mu = _allsum(sv, perms) * (1.0 / DIM)
                var = _allsum(qv, perms) * (1.0 / DIM) - mu * mu
                rv = _rsqrt(var + EPS)
                for k in range(NV):
                    ro_b[i, pl.ds(16 * k, 16)] = \
                        (hs[k] - mu) * rv * g[k] + b[k]

    def pair_body(i, carry):
        wait_gathers(0)
        issue(2 * i + 1, 1)
        compute(0)
        pltpu.sync_copy(ro[0], out_h.at[pl.ds((wid * CNK + 2 * i) * CR, CR)])

        wait_gathers(1)

        @pl.when(i < NPAIR - 1)
        def _():
            issue(2 * i + 2, 0)

        compute(1)
        pltpu.sync_copy(ro[1],
                        out_h.at[pl.ds((wid * CNK + 2 * i + 1) * CR, CR)])
        return carry

    issue(0, 0)
    lax.fori_loop(0, NPAIR, pair_body, 0, unroll=False)


_emb_ln = functools.partial(
    pl.kernel,
    out_type=jax.ShapeDtypeStruct((N, DIM), jnp.float32),
    mesh=plsc.VectorSubcoreMesh(core_axis_name="c", subcore_axis_name="s"),
    compiler_params=pltpu.CompilerParams(use_tc_tiling_on_sc=False),
    scratch_types=[
        pltpu.VMEM((CR,), jnp.int32),
        pltpu.VMEM((CR,), jnp.int32),
        pltpu.VMEM((CR,), jnp.int32),
        pltpu.VMEM((CR,), jnp.int32),
        pltpu.VMEM((CR,), jnp.int32),
        pltpu.VMEM((CR,), jnp.int32),
        pltpu.VMEM((CR, DIM), jnp.float32),
        pltpu.VMEM((CR, DIM), jnp.float32),
        pltpu.VMEM((CR, DIM), jnp.float32),
        pltpu.VMEM((CR, DIM), jnp.float32),
        pltpu.VMEM((CR, DIM), jnp.float32),
        pltpu.VMEM((DIM,), jnp.float32),
        pltpu.VMEM((DIM,), jnp.float32),
        pltpu.VMEM((SEGN * DIM,), jnp.float32),
        pltpu.SemaphoreType.DMA,
        pltpu.SemaphoreType.DMA,
        pltpu.SemaphoreType.DMA,
        pltpu.SemaphoreType.DMA,
    ],
)(_body)


@jax.jit
def kernel(emb, posemb, segemb, gamma, beta, x, seg, pos):
    embs = jnp.concatenate([emb[:HV], emb[HV:]], axis=1)
    emb64 = embs.reshape(VOC, DIM)
    xg = jnp.where(x < HV, 2 * x, 2 * (x - HV) + 1).reshape(N)
    sf = seg.reshape(N)
    pf = pos.reshape(N)
    out = _emb_ln(emb64, posemb, segemb.reshape(SEGN * DIM),
                  gamma, beta, xg, sf, pf)
    return out.reshape(B, L, DIM)
